# baseline (speedup 1.0000x reference)
import functools

import jax
import jax.numpy as jnp
from jax.experimental import pallas as pl
from jax.experimental.pallas import tpu as pltpu
from jax.experimental.pallas import tpu_sc as plsc

_T = 2048
_E = 8
_NG = 4
_GS = _E // _NG
_GCAP = 2048
_NSLOT = _NG * _GCAP
_TBG = 512
_D = 1536
_NW = 32
_BPW = _T // _NW
_CH = 16
_NCH = _BPW // _CH


def _dot(a, b):
    return jax.lax.dot_general(
        a, b, (((1,), (1,)), ((), ())), preferred_element_type=jnp.float32)


def _bdot(a, b):
    return jax.lax.dot_general(
        a.astype(jnp.bfloat16), b.astype(jnp.bfloat16),
        (((1,), (1,)), ((), ())), preferred_element_type=jnp.float32)


def _silu(v):
    return v * jax.nn.sigmoid(v)


def _routing_scores_sel(z, gate):
    logits = _dot(z, gate)
    m = jnp.max(logits, axis=1, keepdims=True)
    p = jnp.exp(logits - m)
    scores = p / jnp.sum(p, axis=1, keepdims=True)
    lane = jax.lax.broadcasted_iota(jnp.int32, scores.shape, 1)
    cols = [
        jnp.sum(jnp.where(lane == j, scores, 0.0), axis=1, keepdims=True)
        for j in range(_E)
    ]
    gsc = [jnp.maximum(cols[2 * g], cols[2 * g + 1]) for g in range(_NG)]
    sel = []
    for g in range(_NG):
        rank = jnp.zeros_like(gsc[g])
        for j in range(_NG):
            if j == g:
                continue
            beats = (gsc[j] > gsc[g]) | ((gsc[j] == gsc[g]) & (j < g))
            rank = rank + beats.astype(jnp.float32)
        sel.append((rank < 2.0).astype(jnp.float32))
    return scores, sel


def _combine_from(scores, sel):
    lane = jax.lax.broadcasted_iota(jnp.int32, scores.shape, 1)
    emask = jnp.zeros_like(scores)
    for g in range(_NG):
        emask = emask + jnp.where(lane // _GS == g, sel[g], 0.0)
    return scores * emask


def _small_layer_body(dout, x_ref, wl_ref, bl_ref, ws_ref, bs_ref, gate_ref,
                      w1_ref, w3_ref, w2_ref, s1_ref, s3_ref, s2_ref, out_ref):
    x = x_ref[...]
    y = _dot(x, wl_ref[...]) + bl_ref[...]
    h = _dot(y, ws_ref[...]) + bs_ref[...]
    u = h[:, :dout]
    v = h[:, dout:]
    z = u * _silu(v)
    scores, sel = _routing_scores_sel(z, gate_ref[...])
    combine = _combine_from(scores, sel)
    a1 = _dot(z, s1_ref[...])
    a3 = _dot(z, s3_ref[...])
    acc = _dot(_silu(a1) * a3, s2_ref[...])
    for e in range(_E):
        h1 = _dot(z, w1_ref[e])
        h3 = _dot(z, w3_ref[e])
        eo = _dot(_silu(h1) * h3, w2_ref[e])
        w = jnp.sum(
            jnp.where(jax.lax.broadcasted_iota(jnp.int32, combine.shape, 1) == e,
                      combine, 0.0), axis=1, keepdims=True)
        acc = acc + w * eo
    out_ref[...] = acc


def _moeff_small(p, x, din, dout, tb=512):
    hdim = dout // 2
    hs = dout
    moe = p["moe"]
    full = lambda shp: pl.BlockSpec(shp, lambda i: (0,) * len(shp))
    return pl.pallas_call(
        functools.partial(_small_layer_body, dout),
        grid=(_T // tb,),
        in_specs=[
            pl.BlockSpec((tb, din), lambda i: (i, 0)),
            full((dout, din)), full((1, dout)),
            full((2 * dout, dout)), full((1, 2 * dout)),
            full((_E, dout)),
            full((_E, hdim, dout)), full((_E, hdim, dout)),
            full((_E, dout, hdim)),
            full((hs, dout)), full((hs, dout)), full((dout, hs)),
        ],
        out_specs=pl.BlockSpec((tb, dout), lambda i: (i, 0)),
        out_shape=jax.ShapeDtypeStruct((_T, dout), jnp.float32),
    )(x, p["lin"]["W"], p["lin"]["b"].reshape(1, -1),
      p["sw"]["W"], p["sw"]["b"].reshape(1, -1), moe["gate"],
      moe["w1"], moe["w3"], moe["w2"],
      moe["sw1"], moe["sw3"], moe["sw2"])


def _big_prelude_body(dout, tb, x_ref, wl_ref, bl_ref, ws_ref, bs_ref,
                      gate_ref,
                      z_ref, slotA_ref, slotB_ref,
                      wA_ref, wB_ref, cnt_out_ref, cnt_ref):
    @pl.when(pl.program_id(0) == 0)
    def _():
        cnt_ref[...] = jnp.zeros_like(cnt_ref)

    x = x_ref[...]
    y = _dot(x, wl_ref[...]) + bl_ref[...]
    h = _dot(y, ws_ref[...]) + bs_ref[...]
    u = h[:, :dout]
    v = h[:, dout:]
    z = u * _silu(v)
    scores, sel = _routing_scores_sel(z, gate_ref[...])

    lane8 = jax.lax.broadcasted_iota(jnp.int32, scores.shape, 1)
    lane8f = lane8.astype(jnp.float32)
    sel8 = jnp.zeros_like(scores)
    for g in range(_NG):
        sel8 = sel8 + jnp.where(lane8 == g, sel[g], 0.0)
    row_i = jax.lax.broadcasted_iota(jnp.int32, (tb, tb), 0)
    col_i = jax.lax.broadcasted_iota(jnp.int32, (tb, tb), 1)
    ltri = (row_i > col_i).astype(jnp.float32)
    ranks = jax.lax.dot_general(
        ltri, sel8, (((1,), (0,)), ((), ())),
        preferred_element_type=jnp.float32)
    ranks = ranks + cnt_ref[...]
    gAf = jnp.min(jnp.where(sel8 > 0, lane8f, 1e9), axis=1, keepdims=True)
    gBf = jnp.max(jnp.where(sel8 > 0, lane8f, -1.0), axis=1, keepdims=True)
    rankA = jnp.sum(jnp.where(lane8f == gAf, ranks, 0.0), axis=1, keepdims=True)
    rankB = jnp.sum(jnp.where(lane8f == gBf, ranks, 0.0), axis=1, keepdims=True)
    sA = jnp.round(gAf * _GCAP + rankA).astype(jnp.int32)
    sB = jnp.round(gBf * _GCAP + rankB).astype(jnp.int32)
    tb_ = sA.shape[0]
    slotA_ref[...] = sA
    slotB_ref[...] = sB
    lane128 = jax.lax.broadcasted_iota(jnp.int32, (tb_, 128), 1)
    for (gf, w_ref) in ((gAf, wA_ref), (gBf, wB_ref)):
        w0 = jnp.sum(jnp.where(lane8f == 2.0 * gf, scores, 0.0),
                     axis=1, keepdims=True)
        w1 = jnp.sum(jnp.where(lane8f == 2.0 * gf + 1.0, scores, 0.0),
                     axis=1, keepdims=True)
        w_ref[...] = (w0 * (lane128 == 0) + w1 * (lane128 == 1)
                      ).astype(jnp.float32)
    cnt = cnt_ref[...] + jnp.sum(sel8, axis=0, keepdims=True)
    cnt_ref[...] = cnt
    cnt_out_ref[...] = jnp.round(cnt).astype(jnp.int32)

    z_ref[...] = z


def _shared_body(z_ref, s1_ref, s3_ref, s2_ref, out_ref):
    zb = z_ref[...].astype(jnp.bfloat16)
    a1 = _bdot(zb, s1_ref[...])
    a3 = _bdot(zb, s3_ref[...])
    out_ref[...] = _bdot(_silu(a1) * a3, s2_ref[...])


def _grouped_body(cnt_ref, z_ref, wp_ref, w1_ref, w3_ref, w2_ref, out_ref):
    i = pl.program_id(0)
    g = i // (_GCAP // _TBG)
    start = (i % (_GCAP // _TBG)) * _TBG

    @pl.when(start < cnt_ref[g])
    def _():
        zs = z_ref[...].astype(jnp.bfloat16)
        wp = wp_ref[...]
        lane = jax.lax.broadcasted_iota(jnp.int32, wp.shape, 1)
        w0 = jnp.sum(jnp.where(lane == 0, wp, 0.0), axis=1, keepdims=True)
        w1 = jnp.sum(jnp.where(lane == 1, wp, 0.0), axis=1, keepdims=True)
        h1a = _bdot(zs, w1_ref[0, 0])
        h3a = _bdot(zs, w3_ref[0, 0])
        eoa = _bdot(_silu(h1a) * h3a, w2_ref[0, 0])
        h1b = _bdot(zs, w1_ref[0, 1])
        h3b = _bdot(zs, w3_ref[0, 1])
        eob = _bdot(_silu(h1b) * h3b, w2_ref[0, 1])
        out_ref[...] = w0 * eoa + w1 * eob


def _vmesh():
    return plsc.VectorSubcoreMesh(core_axis_name="core",
                                  subcore_axis_name="subcore")


def _sc_dispatch(z, iA3, iB3, wA, wB):
    out_types = [jax.ShapeDtypeStruct((_NSLOT, _D), jnp.float32),
                 jax.ShapeDtypeStruct((_NSLOT, 128), jnp.float32)]

    @pl.kernel(out_type=out_types, mesh=_vmesh(),
               scratch_types=[pltpu.VMEM((_NCH, _CH), jnp.int32),
                              pltpu.VMEM((_NCH, _CH), jnp.int32),
                              pltpu.VMEM((_CH, _D), jnp.float32),
                              pltpu.VMEM((_CH, 128), jnp.float32),
                              pltpu.VMEM((_CH, 128), jnp.float32),
                              pltpu.SemaphoreType.DMA])
    def k(z_hbm, iA3_hbm, iB3_hbm, wA_hbm, wB_hbm, zs_hbm, wp_hbm,
          idxA_v, idxB_v, zrow_v, wrow_v, wrow2_v, sem):
        wid = (jax.lax.axis_index("subcore") * 2
               + jax.lax.axis_index("core"))
        pltpu.sync_copy(iA3_hbm.at[wid], idxA_v)
        pltpu.sync_copy(iB3_hbm.at[wid], idxB_v)
        for c in range(_NCH):
            base = wid * _BPW + c * _CH
            pltpu.sync_copy(z_hbm.at[pl.ds(base, _CH)], zrow_v)
            cp1 = pltpu.async_copy(zrow_v, zs_hbm.at[idxA_v.at[c]], sem)
            cp2 = pltpu.async_copy(zrow_v, zs_hbm.at[idxB_v.at[c]], sem)
            pltpu.sync_copy(wA_hbm.at[pl.ds(base, _CH)], wrow_v)
            cp3 = pltpu.async_copy(wrow_v, wp_hbm.at[idxA_v.at[c]], sem)
            pltpu.sync_copy(wB_hbm.at[pl.ds(base, _CH)], wrow2_v)
            cp4 = pltpu.async_copy(wrow2_v, wp_hbm.at[idxB_v.at[c]], sem)
            cp1.wait()
            cp2.wait()
            cp3.wait()
            cp4.wait()

    return k(z, iA3, iB3, wA, wB)


def _sc_combine(out_sorted, iA3, iB3, shared):
    @pl.kernel(out_type=jax.ShapeDtypeStruct((_T, _D), jnp.float32),
               mesh=_vmesh(),
               scratch_types=[pltpu.VMEM((_NCH, _CH), jnp.int32),
                              pltpu.VMEM((_NCH, _CH), jnp.int32),
                              pltpu.VMEM((_CH, _D), jnp.float32),
                              pltpu.VMEM((_CH, _D), jnp.float32),
                              pltpu.VMEM((_CH, _D), jnp.float32),
                              pltpu.VMEM((_CH, _D), jnp.float32),
                              pltpu.VMEM((_CH, _D), jnp.float32),
                              pltpu.SemaphoreType.DMA,
                              pltpu.SemaphoreType.DMA])
    def k(os_hbm, iA3_hbm, iB3_hbm, sh_hbm, out_hbm,
          idxA_v, idxB_v, bufA0, bufB0, bufA1, bufB1, acc_v, semA, semB):
        wid = (jax.lax.axis_index("subcore") * 2
               + jax.lax.axis_index("core"))
        pltpu.sync_copy(iA3_hbm.at[wid], idxA_v)
        pltpu.sync_copy(iB3_hbm.at[wid], idxB_v)
        bufs = [(bufA0, bufB0), (bufA1, bufB1)]
        pend = [None, None]
        pend[0] = (pltpu.async_copy(os_hbm.at[idxA_v.at[0]], bufA0, semA),
                   pltpu.async_copy(os_hbm.at[idxB_v.at[0]], bufB0, semB))
        for c in range(_NCH):
            cur = c % 2
            if c + 1 < _NCH:
                nA, nB = bufs[(c + 1) % 2]
                pend[(c + 1) % 2] = (
                    pltpu.async_copy(os_hbm.at[idxA_v.at[c + 1]], nA, semA),
                    pltpu.async_copy(os_hbm.at[idxB_v.at[c + 1]], nB, semB))
            base = wid * _BPW + c * _CH
            pltpu.sync_copy(sh_hbm.at[pl.ds(base, _CH)], acc_v)
            cpA, cpB = pend[cur]
            cpA.wait()
            cpB.wait()
            bA, bB = bufs[cur]

            @pl.loop(0, _CH)
            def _(r):
                @pl.loop(0, _D, step=16)
                def _(col):
                    slc = (pl.ds(r, 1), pl.ds(col, 16))
                    acc_v.at[*slc][...] = (acc_v.at[*slc][...]
                                           + bA.at[*slc][...]
                                           + bB.at[*slc][...])

            pltpu.sync_copy(acc_v, out_hbm.at[pl.ds(base, _CH)])

    return k(out_sorted, iA3, iB3, shared)


def _moeff_big(p, x):
    din, dout, hdim = 512, 1536, 768
    hs = dout
    moe = p["moe"]
    full = lambda shp: pl.BlockSpec(shp, lambda i: (0,) * len(shp))
    tb = 256
    nb = _T // tb
    (z, slotA, slotB, wA, wB, counts) = pl.pallas_call(
        functools.partial(_big_prelude_body, dout, tb),
        grid=(nb,),
        in_specs=[
            pl.BlockSpec((tb, din), lambda i: (i, 0)),
            full((dout, din)), full((1, dout)),
            full((2 * dout, dout)), full((1, 2 * dout)),
            full((_E, dout)),
        ],
        out_specs=[
            pl.BlockSpec((tb, dout), lambda i: (i, 0)),
            pl.BlockSpec((tb, 1), lambda i: (i, 0)),
            pl.BlockSpec((tb, 1), lambda i: (i, 0)),
            pl.BlockSpec((tb, 128), lambda i: (i, 0)),
            pl.BlockSpec((tb, 128), lambda i: (i, 0)),
            pl.BlockSpec((1, _E), lambda i: (0, 0)),
        ],
        out_shape=[
            jax.ShapeDtypeStruct((_T, dout), jnp.float32),
            jax.ShapeDtypeStruct((_T, 1), jnp.int32),
            jax.ShapeDtypeStruct((_T, 1), jnp.int32),
            jax.ShapeDtypeStruct((_T, 128), jnp.float32),
            jax.ShapeDtypeStruct((_T, 128), jnp.float32),
            jax.ShapeDtypeStruct((1, _E), jnp.int32),
        ],
        scratch_shapes=[pltpu.VMEM((1, _E), jnp.float32)],
    )(x, p["lin"]["W"], p["lin"]["b"].reshape(1, -1),
      p["sw"]["W"], p["sw"]["b"].reshape(1, -1), moe["gate"])

    iA3 = slotA.reshape(_NW, _NCH, _CH)
    iB3 = slotB.reshape(_NW, _NCH, _CH)
    zs, wp = _sc_dispatch(z, iA3, iB3, wA, wB)

    stb = 1024
    shared = pl.pallas_call(
        _shared_body,
        grid=(_T // stb,),
        in_specs=[
            pl.BlockSpec((stb, dout), lambda i: (i, 0)),
            full((hs, dout)), full((hs, dout)), full((dout, hs)),
        ],
        out_specs=pl.BlockSpec((stb, dout), lambda i: (i, 0)),
        out_shape=jax.ShapeDtypeStruct((_T, dout), jnp.float32),
    )(z, moe["sw1"].astype(jnp.bfloat16), moe["sw3"].astype(jnp.bfloat16),
      moe["sw2"].astype(jnp.bfloat16))

    w1g = moe["w1"].astype(jnp.bfloat16).reshape(_NG, _GS, hdim, dout)
    w3g = moe["w3"].astype(jnp.bfloat16).reshape(_NG, _GS, hdim, dout)
    w2g = moe["w2"].astype(jnp.bfloat16).reshape(_NG, _GS, dout, hdim)
    nblk = _NSLOT // _TBG
    out_sorted = pl.pallas_call(
        _grouped_body,
        grid_spec=pltpu.PrefetchScalarGridSpec(
            num_scalar_prefetch=1,
            grid=(nblk,),
            in_specs=[
                pl.BlockSpec((_TBG, dout), lambda i, c: (i, 0)),
                pl.BlockSpec((_TBG, 128), lambda i, c: (i, 0)),
                pl.BlockSpec((1, _GS, hdim, dout),
                             lambda i, c: (i // (_GCAP // _TBG), 0, 0, 0)),
                pl.BlockSpec((1, _GS, hdim, dout),
                             lambda i, c: (i // (_GCAP // _TBG), 0, 0, 0)),
                pl.BlockSpec((1, _GS, dout, hdim),
                             lambda i, c: (i // (_GCAP // _TBG), 0, 0, 0)),
            ],
            out_specs=pl.BlockSpec((_TBG, dout), lambda i, c: (i, 0)),
        ),
        out_shape=jax.ShapeDtypeStruct((_NSLOT, dout), jnp.float32),
    )(counts.reshape(_E), zs, wp, w1g, w3g, w2g)

    return _sc_combine(out_sorted, iA3, iB3, shared)


def kernel(x, params):
    skip = _moeff_small(params["enc0"], x, 768, 256)
    b = _moeff_small(params["bott"], skip, 256, 256)
    d = jnp.concatenate([b, skip], axis=1)
    return _moeff_big(params["dec0"], d)


# speedup vs baseline: 1.0317x; 1.0317x over previous
import functools

import jax
import jax.numpy as jnp
from jax.experimental import pallas as pl
from jax.experimental.pallas import tpu as pltpu
from jax.experimental.pallas import tpu_sc as plsc

_T = 2048
_E = 8
_NG = 4
_GS = _E // _NG
_GCAP = 2048
_NSLOT = _NG * _GCAP
_TBG = 512
_D = 1536
_NW = 32
_BPW = _T // _NW
_CH = 16
_NCH = _BPW // _CH


def _dot(a, b):
    return jax.lax.dot_general(
        a, b, (((1,), (1,)), ((), ())), preferred_element_type=jnp.float32)


def _bdot(a, b):
    return jax.lax.dot_general(
        a.astype(jnp.bfloat16), b.astype(jnp.bfloat16),
        (((1,), (1,)), ((), ())), preferred_element_type=jnp.float32)


def _silu(v):
    return v * jax.nn.sigmoid(v)


def _routing_scores_sel(z, gate):
    logits = _dot(z, gate)
    m = jnp.max(logits, axis=1, keepdims=True)
    p = jnp.exp(logits - m)
    scores = p / jnp.sum(p, axis=1, keepdims=True)
    lane = jax.lax.broadcasted_iota(jnp.int32, scores.shape, 1)
    cols = [
        jnp.sum(jnp.where(lane == j, scores, 0.0), axis=1, keepdims=True)
        for j in range(_E)
    ]
    gsc = [jnp.maximum(cols[2 * g], cols[2 * g + 1]) for g in range(_NG)]
    sel = []
    for g in range(_NG):
        rank = jnp.zeros_like(gsc[g])
        for j in range(_NG):
            if j == g:
                continue
            beats = (gsc[j] > gsc[g]) | ((gsc[j] == gsc[g]) & (j < g))
            rank = rank + beats.astype(jnp.float32)
        sel.append((rank < 2.0).astype(jnp.float32))
    return scores, sel


def _combine_from(scores, sel):
    lane = jax.lax.broadcasted_iota(jnp.int32, scores.shape, 1)
    emask = jnp.zeros_like(scores)
    for g in range(_NG):
        emask = emask + jnp.where(lane // _GS == g, sel[g], 0.0)
    return scores * emask


def _small_layer_body(dout, x_ref, wl_ref, bl_ref, ws_ref, bs_ref, gate_ref,
                      w1_ref, w3_ref, w2_ref, s1_ref, s3_ref, s2_ref, out_ref):
    x = x_ref[...]
    y = _dot(x, wl_ref[...]) + bl_ref[...]
    h = _dot(y, ws_ref[...]) + bs_ref[...]
    u = h[:, :dout]
    v = h[:, dout:]
    z = u * _silu(v)
    scores, sel = _routing_scores_sel(z, gate_ref[...])
    combine = _combine_from(scores, sel)
    a1 = _dot(z, s1_ref[...])
    a3 = _dot(z, s3_ref[...])
    acc = _dot(_silu(a1) * a3, s2_ref[...])
    for e in range(_E):
        h1 = _dot(z, w1_ref[e])
        h3 = _dot(z, w3_ref[e])
        eo = _dot(_silu(h1) * h3, w2_ref[e])
        w = jnp.sum(
            jnp.where(jax.lax.broadcasted_iota(jnp.int32, combine.shape, 1) == e,
                      combine, 0.0), axis=1, keepdims=True)
        acc = acc + w * eo
    out_ref[...] = acc


def _moeff_small(p, x, din, dout, tb=512):
    hdim = dout // 2
    hs = dout
    moe = p["moe"]
    full = lambda shp: pl.BlockSpec(shp, lambda i: (0,) * len(shp))
    return pl.pallas_call(
        functools.partial(_small_layer_body, dout),
        grid=(_T // tb,),
        in_specs=[
            pl.BlockSpec((tb, din), lambda i: (i, 0)),
            full((dout, din)), full((1, dout)),
            full((2 * dout, dout)), full((1, 2 * dout)),
            full((_E, dout)),
            full((_E, hdim, dout)), full((_E, hdim, dout)),
            full((_E, dout, hdim)),
            full((hs, dout)), full((hs, dout)), full((dout, hs)),
        ],
        out_specs=pl.BlockSpec((tb, dout), lambda i: (i, 0)),
        out_shape=jax.ShapeDtypeStruct((_T, dout), jnp.float32),
    )(x, p["lin"]["W"], p["lin"]["b"].reshape(1, -1),
      p["sw"]["W"], p["sw"]["b"].reshape(1, -1), moe["gate"],
      moe["w1"], moe["w3"], moe["w2"],
      moe["sw1"], moe["sw3"], moe["sw2"])


def _big_prelude_body(dout, tb, x_ref, wl_ref, bl_ref, ws_ref, bs_ref,
                      gate_ref,
                      z_ref, slotA_ref, slotB_ref,
                      wA_ref, wB_ref, cnt_out_ref, cnt_ref):
    @pl.when(pl.program_id(0) == 0)
    def _():
        cnt_ref[...] = jnp.zeros_like(cnt_ref)

    x = x_ref[...]
    y = _dot(x, wl_ref[...]) + bl_ref[...]
    h = _dot(y, ws_ref[...]) + bs_ref[...]
    u = h[:, :dout]
    v = h[:, dout:]
    z = u * _silu(v)
    scores, sel = _routing_scores_sel(z, gate_ref[...])

    lane8 = jax.lax.broadcasted_iota(jnp.int32, scores.shape, 1)
    lane8f = lane8.astype(jnp.float32)
    sel8 = jnp.zeros_like(scores)
    for g in range(_NG):
        sel8 = sel8 + jnp.where(lane8 == g, sel[g], 0.0)
    row_i = jax.lax.broadcasted_iota(jnp.int32, (tb, tb), 0)
    col_i = jax.lax.broadcasted_iota(jnp.int32, (tb, tb), 1)
    ltri = (row_i > col_i).astype(jnp.float32)
    ranks = jax.lax.dot_general(
        ltri, sel8, (((1,), (0,)), ((), ())),
        preferred_element_type=jnp.float32)
    ranks = ranks + cnt_ref[...]
    gAf = jnp.min(jnp.where(sel8 > 0, lane8f, 1e9), axis=1, keepdims=True)
    gBf = jnp.max(jnp.where(sel8 > 0, lane8f, -1.0), axis=1, keepdims=True)
    rankA = jnp.sum(jnp.where(lane8f == gAf, ranks, 0.0), axis=1, keepdims=True)
    rankB = jnp.sum(jnp.where(lane8f == gBf, ranks, 0.0), axis=1, keepdims=True)
    sA = jnp.round(gAf * _GCAP + rankA).astype(jnp.int32)
    sB = jnp.round(gBf * _GCAP + rankB).astype(jnp.int32)
    tb_ = sA.shape[0]
    slotA_ref[...] = sA
    slotB_ref[...] = sB
    lane128 = jax.lax.broadcasted_iota(jnp.int32, (tb_, 128), 1)
    for (gf, w_ref) in ((gAf, wA_ref), (gBf, wB_ref)):
        w0 = jnp.sum(jnp.where(lane8f == 2.0 * gf, scores, 0.0),
                     axis=1, keepdims=True)
        w1 = jnp.sum(jnp.where(lane8f == 2.0 * gf + 1.0, scores, 0.0),
                     axis=1, keepdims=True)
        w_ref[...] = (w0 * (lane128 == 0) + w1 * (lane128 == 1)
                      ).astype(jnp.float32)
    cnt = cnt_ref[...] + jnp.sum(sel8, axis=0, keepdims=True)
    cnt_ref[...] = cnt
    cnt_out_ref[...] = jnp.round(cnt).astype(jnp.int32)

    z_ref[...] = z


def _shared_body(z_ref, s1_ref, s3_ref, s2_ref, out_ref):
    z = z_ref[...]
    a1 = _dot(z, s1_ref[...])
    a3 = _dot(z, s3_ref[...])
    out_ref[...] = _dot(_silu(a1) * a3, s2_ref[...])


def _grouped_body(cnt_ref, z_ref, wp_ref, w1_ref, w3_ref, w2_ref, out_ref):
    i = pl.program_id(0)
    g = i // (_GCAP // _TBG)
    start = (i % (_GCAP // _TBG)) * _TBG

    @pl.when(start < cnt_ref[g])
    def _():
        zs = z_ref[...].astype(jnp.bfloat16)
        wp = wp_ref[...]
        lane = jax.lax.broadcasted_iota(jnp.int32, wp.shape, 1)
        w0 = jnp.sum(jnp.where(lane == 0, wp, 0.0), axis=1, keepdims=True)
        w1 = jnp.sum(jnp.where(lane == 1, wp, 0.0), axis=1, keepdims=True)
        h1a = _bdot(zs, w1_ref[0, 0])
        h3a = _bdot(zs, w3_ref[0, 0])
        eoa = _bdot(_silu(h1a) * h3a, w2_ref[0, 0])
        h1b = _bdot(zs, w1_ref[0, 1])
        h3b = _bdot(zs, w3_ref[0, 1])
        eob = _bdot(_silu(h1b) * h3b, w2_ref[0, 1])
        out_ref[...] = w0 * eoa + w1 * eob


def _vmesh():
    return plsc.VectorSubcoreMesh(core_axis_name="core",
                                  subcore_axis_name="subcore")


def _sc_dispatch(z, iA3, iB3, wA, wB):
    out_types = [jax.ShapeDtypeStruct((_NSLOT, _D), jnp.float32),
                 jax.ShapeDtypeStruct((_NSLOT, 128), jnp.float32)]

    @pl.kernel(out_type=out_types, mesh=_vmesh(),
               scratch_types=[pltpu.VMEM((_NCH, _CH), jnp.int32),
                              pltpu.VMEM((_NCH, _CH), jnp.int32),
                              pltpu.VMEM((_CH, _D), jnp.float32),
                              pltpu.VMEM((_CH, 128), jnp.float32),
                              pltpu.VMEM((_CH, 128), jnp.float32),
                              pltpu.SemaphoreType.DMA])
    def k(z_hbm, iA3_hbm, iB3_hbm, wA_hbm, wB_hbm, zs_hbm, wp_hbm,
          idxA_v, idxB_v, zrow_v, wrow_v, wrow2_v, sem):
        wid = (jax.lax.axis_index("subcore") * 2
               + jax.lax.axis_index("core"))
        pltpu.sync_copy(iA3_hbm.at[wid], idxA_v)
        pltpu.sync_copy(iB3_hbm.at[wid], idxB_v)
        for c in range(_NCH):
            base = wid * _BPW + c * _CH
            pltpu.sync_copy(z_hbm.at[pl.ds(base, _CH)], zrow_v)
            cp1 = pltpu.async_copy(zrow_v, zs_hbm.at[idxA_v.at[c]], sem)
            cp2 = pltpu.async_copy(zrow_v, zs_hbm.at[idxB_v.at[c]], sem)
            pltpu.sync_copy(wA_hbm.at[pl.ds(base, _CH)], wrow_v)
            cp3 = pltpu.async_copy(wrow_v, wp_hbm.at[idxA_v.at[c]], sem)
            pltpu.sync_copy(wB_hbm.at[pl.ds(base, _CH)], wrow2_v)
            cp4 = pltpu.async_copy(wrow2_v, wp_hbm.at[idxB_v.at[c]], sem)
            cp1.wait()
            cp2.wait()
            cp3.wait()
            cp4.wait()

    return k(z, iA3, iB3, wA, wB)


def _sc_combine(out_sorted, iA3, iB3, shared):
    @pl.kernel(out_type=jax.ShapeDtypeStruct((_T, _D), jnp.float32),
               mesh=_vmesh(),
               scratch_types=[pltpu.VMEM((_NCH, _CH), jnp.int32),
                              pltpu.VMEM((_NCH, _CH), jnp.int32),
                              pltpu.VMEM((_CH, _D), jnp.float32),
                              pltpu.VMEM((_CH, _D), jnp.float32),
                              pltpu.VMEM((_CH, _D), jnp.float32),
                              pltpu.VMEM((_CH, _D), jnp.float32),
                              pltpu.VMEM((_CH, _D), jnp.float32),
                              pltpu.SemaphoreType.DMA,
                              pltpu.SemaphoreType.DMA])
    def k(os_hbm, iA3_hbm, iB3_hbm, sh_hbm, out_hbm,
          idxA_v, idxB_v, bufA0, bufB0, bufA1, bufB1, acc_v, semA, semB):
        wid = (jax.lax.axis_index("subcore") * 2
               + jax.lax.axis_index("core"))
        pltpu.sync_copy(iA3_hbm.at[wid], idxA_v)
        pltpu.sync_copy(iB3_hbm.at[wid], idxB_v)
        bufs = [(bufA0, bufB0), (bufA1, bufB1)]
        pend = [None, None]
        pend[0] = (pltpu.async_copy(os_hbm.at[idxA_v.at[0]], bufA0, semA),
                   pltpu.async_copy(os_hbm.at[idxB_v.at[0]], bufB0, semB))
        for c in range(_NCH):
            cur = c % 2
            if c + 1 < _NCH:
                nA, nB = bufs[(c + 1) % 2]
                pend[(c + 1) % 2] = (
                    pltpu.async_copy(os_hbm.at[idxA_v.at[c + 1]], nA, semA),
                    pltpu.async_copy(os_hbm.at[idxB_v.at[c + 1]], nB, semB))
            base = wid * _BPW + c * _CH
            pltpu.sync_copy(sh_hbm.at[pl.ds(base, _CH)], acc_v)
            cpA, cpB = pend[cur]
            cpA.wait()
            cpB.wait()
            bA, bB = bufs[cur]

            @pl.loop(0, _CH)
            def _(r):
                @pl.loop(0, _D, step=16)
                def _(col):
                    slc = (pl.ds(r, 1), pl.ds(col, 16))
                    acc_v.at[*slc][...] = (acc_v.at[*slc][...]
                                           + bA.at[*slc][...]
                                           + bB.at[*slc][...])

            pltpu.sync_copy(acc_v, out_hbm.at[pl.ds(base, _CH)])

    return k(out_sorted, iA3, iB3, shared)


def _moeff_big(p, x):
    din, dout, hdim = 512, 1536, 768
    hs = dout
    moe = p["moe"]
    full = lambda shp: pl.BlockSpec(shp, lambda i: (0,) * len(shp))
    tb = 256
    nb = _T // tb
    (z, slotA, slotB, wA, wB, counts) = pl.pallas_call(
        functools.partial(_big_prelude_body, dout, tb),
        grid=(nb,),
        in_specs=[
            pl.BlockSpec((tb, din), lambda i: (i, 0)),
            full((dout, din)), full((1, dout)),
            full((2 * dout, dout)), full((1, 2 * dout)),
            full((_E, dout)),
        ],
        out_specs=[
            pl.BlockSpec((tb, dout), lambda i: (i, 0)),
            pl.BlockSpec((tb, 1), lambda i: (i, 0)),
            pl.BlockSpec((tb, 1), lambda i: (i, 0)),
            pl.BlockSpec((tb, 128), lambda i: (i, 0)),
            pl.BlockSpec((tb, 128), lambda i: (i, 0)),
            pl.BlockSpec((1, _E), lambda i: (0, 0)),
        ],
        out_shape=[
            jax.ShapeDtypeStruct((_T, dout), jnp.float32),
            jax.ShapeDtypeStruct((_T, 1), jnp.int32),
            jax.ShapeDtypeStruct((_T, 1), jnp.int32),
            jax.ShapeDtypeStruct((_T, 128), jnp.float32),
            jax.ShapeDtypeStruct((_T, 128), jnp.float32),
            jax.ShapeDtypeStruct((1, _E), jnp.int32),
        ],
        scratch_shapes=[pltpu.VMEM((1, _E), jnp.float32)],
    )(x, p["lin"]["W"], p["lin"]["b"].reshape(1, -1),
      p["sw"]["W"], p["sw"]["b"].reshape(1, -1), moe["gate"])

    iA3 = slotA.reshape(_NW, _NCH, _CH)
    iB3 = slotB.reshape(_NW, _NCH, _CH)
    zs, wp = _sc_dispatch(z, iA3, iB3, wA, wB)

    stb = 512
    shared = pl.pallas_call(
        _shared_body,
        grid=(_T // stb,),
        in_specs=[
            pl.BlockSpec((stb, dout), lambda i: (i, 0)),
            full((hs, dout)), full((hs, dout)), full((dout, hs)),
        ],
        out_specs=pl.BlockSpec((stb, dout), lambda i: (i, 0)),
        out_shape=jax.ShapeDtypeStruct((_T, dout), jnp.float32),
    )(z, moe["sw1"], moe["sw3"], moe["sw2"])

    w1g = moe["w1"].astype(jnp.bfloat16).reshape(_NG, _GS, hdim, dout)
    w3g = moe["w3"].astype(jnp.bfloat16).reshape(_NG, _GS, hdim, dout)
    w2g = moe["w2"].astype(jnp.bfloat16).reshape(_NG, _GS, dout, hdim)
    nblk = _NSLOT // _TBG
    out_sorted = pl.pallas_call(
        _grouped_body,
        grid_spec=pltpu.PrefetchScalarGridSpec(
            num_scalar_prefetch=1,
            grid=(nblk,),
            in_specs=[
                pl.BlockSpec((_TBG, dout), lambda i, c: (i, 0)),
                pl.BlockSpec((_TBG, 128), lambda i, c: (i, 0)),
                pl.BlockSpec((1, _GS, hdim, dout),
                             lambda i, c: (i // (_GCAP // _TBG), 0, 0, 0)),
                pl.BlockSpec((1, _GS, hdim, dout),
                             lambda i, c: (i // (_GCAP // _TBG), 0, 0, 0)),
                pl.BlockSpec((1, _GS, dout, hdim),
                             lambda i, c: (i // (_GCAP // _TBG), 0, 0, 0)),
            ],
            out_specs=pl.BlockSpec((_TBG, dout), lambda i, c: (i, 0)),
        ),
        out_shape=jax.ShapeDtypeStruct((_NSLOT, dout), jnp.float32),
    )(counts.reshape(_E), zs, wp, w1g, w3g, w2g)

    return _sc_combine(out_sorted, iA3, iB3, shared)


def kernel(x, params):
    skip = _moeff_small(params["enc0"], x, 768, 256)
    b = _moeff_small(params["bott"], skip, 256, 256)
    d = jnp.concatenate([b, skip], axis=1)
    return _moeff_big(params["dec0"], d)


# speedup vs baseline: 1.3020x; 1.2619x over previous
import functools

import jax
import jax.numpy as jnp
from jax.experimental import pallas as pl
from jax.experimental.pallas import tpu as pltpu

_T = 2048
_E = 8
_NG = 4
_GS = _E // _NG


def _hdot(a, b, prec=None):
    return jax.lax.dot_general(
        a, b, (((1,), (1,)), ((), ())), precision=prec,
        preferred_element_type=jnp.float32)


def _bdot(a, b):
    return jax.lax.dot_general(
        a.astype(jnp.bfloat16), b.astype(jnp.bfloat16),
        (((1,), (1,)), ((), ())), preferred_element_type=jnp.float32)


def _silu(v):
    return v * jax.nn.sigmoid(v)


def _routing_combine(z, gate):
    logits = _hdot(z, gate)
    m = jnp.max(logits, axis=1, keepdims=True)
    p = jnp.exp(logits - m)
    scores = p / jnp.sum(p, axis=1, keepdims=True)
    lane = jax.lax.broadcasted_iota(jnp.int32, scores.shape, 1)
    cols = [
        jnp.sum(jnp.where(lane == j, scores, 0.0), axis=1, keepdims=True)
        for j in range(_E)
    ]
    gsc = [jnp.maximum(cols[2 * g], cols[2 * g + 1]) for g in range(_NG)]
    sel = []
    for g in range(_NG):
        rank = jnp.zeros_like(gsc[g])
        for j in range(_NG):
            if j == g:
                continue
            beats = (gsc[j] > gsc[g]) | ((gsc[j] == gsc[g]) & (j < g))
            rank = rank + beats.astype(jnp.float32)
        sel.append((rank < 2.0).astype(jnp.float32))
    group_lane = lane // _GS
    emask = jnp.zeros_like(scores)
    for g in range(_NG):
        emask = emask + jnp.where(group_lane == g, sel[g], 0.0)
    return scores * emask


def _small_layer_body(dout, x_ref, wl_ref, bl_ref, ws_ref, bs_ref, gate_ref,
                      w1_ref, w3_ref, w2_ref, s1_ref, s3_ref, s2_ref, out_ref):
    x = x_ref[...]
    y = _hdot(x, wl_ref[...]) + bl_ref[...]
    h = _hdot(y, ws_ref[...]) + bs_ref[...]
    u = h[:, :dout]
    v = h[:, dout:]
    z = u * _silu(v)
    combine = _routing_combine(z, gate_ref[...])
    a1 = _hdot(z, s1_ref[...])
    a3 = _hdot(z, s3_ref[...])
    acc = _hdot(_silu(a1) * a3, s2_ref[...])
    for e in range(_E):
        h1 = _hdot(z, w1_ref[e])
        h3 = _hdot(z, w3_ref[e])
        eo = _hdot(_silu(h1) * h3, w2_ref[e])
        w = jnp.sum(
            jnp.where(jax.lax.broadcasted_iota(jnp.int32, combine.shape, 1) == e,
                      combine, 0.0), axis=1, keepdims=True)
        acc = acc + w * eo
    out_ref[...] = acc


def _moeff_small(p, x, din, dout, tb=512):
    hdim = dout // 2
    hs = dout
    moe = p["moe"]
    full = lambda shp: pl.BlockSpec(shp, lambda i: (0,) * len(shp))
    grid = (_T // tb,)
    return pl.pallas_call(
        functools.partial(_small_layer_body, dout),
        grid=grid,
        in_specs=[
            pl.BlockSpec((tb, din), lambda i: (i, 0)),
            full((dout, din)), full((1, dout)),
            full((2 * dout, dout)), full((1, 2 * dout)),
            full((_E, dout)),
            full((_E, hdim, dout)), full((_E, hdim, dout)),
            full((_E, dout, hdim)),
            full((hs, dout)), full((hs, dout)), full((dout, hs)),
        ],
        out_specs=pl.BlockSpec((tb, dout), lambda i: (i, 0)),
        out_shape=jax.ShapeDtypeStruct((_T, dout), jnp.float32),
    )(x, p["lin"]["W"], p["lin"]["b"].reshape(1, -1),
      p["sw"]["W"], p["sw"]["b"].reshape(1, -1), moe["gate"],
      moe["w1"], moe["w3"], moe["w2"],
      moe["sw1"], moe["sw3"], moe["sw2"])


def _big_prelude_body(dout, x_ref, wl_ref, bl_ref, ws_ref, bs_ref, gate_ref,
                      z_ref, comb_ref):
    x = x_ref[...]
    y = _hdot(x, wl_ref[...]) + bl_ref[...]
    h = _hdot(y, ws_ref[...]) + bs_ref[...]
    u = h[:, :dout]
    v = h[:, dout:]
    z = u * _silu(v)
    comb_ref[...] = _routing_combine(z, gate_ref[...])
    z_ref[...] = z.astype(jnp.bfloat16)


def _shared_body(z_ref, s1_ref, s3_ref, s2_ref, out_ref):
    z = z_ref[...].astype(jnp.float32)
    a1 = _hdot(z, s1_ref[...])
    a3 = _hdot(z, s3_ref[...])
    out_ref[...] = _hdot(_silu(a1) * a3, s2_ref[...])


def _big_routed_body(z_ref, comb_ref, shared_ref, w1_ref, w3_ref, w2_ref,
                     out_ref):
    e = pl.program_id(1)

    @pl.when(e == 0)
    def _():
        out_ref[...] = shared_ref[...]

    zb = z_ref[...]
    h1 = _bdot(zb, w1_ref[0])
    h3 = _bdot(zb, w3_ref[0])
    eo = _bdot(_silu(h1) * h3, w2_ref[0])
    comb = comb_ref[...]
    w = jnp.sum(
        jnp.where(jax.lax.broadcasted_iota(jnp.int32, comb.shape, 1) == e,
                  comb, 0.0), axis=1, keepdims=True)
    out_ref[...] = out_ref[...] + w * eo


def _moeff_big(p, x):
    din, dout, hdim = 512, 1536, 768
    hs = dout
    moe = p["moe"]
    full = lambda shp: pl.BlockSpec(shp, lambda i: (0,) * len(shp))
    tb = 512
    z, comb = pl.pallas_call(
        functools.partial(_big_prelude_body, dout),
        grid=(_T // tb,),
        in_specs=[
            pl.BlockSpec((tb, din), lambda i: (i, 0)),
            full((dout, din)), full((1, dout)),
            full((2 * dout, dout)), full((1, 2 * dout)),
            full((_E, dout)),
        ],
        out_specs=[
            pl.BlockSpec((tb, dout), lambda i: (i, 0)),
            pl.BlockSpec((tb, _E), lambda i: (i, 0)),
        ],
        out_shape=[
            jax.ShapeDtypeStruct((_T, dout), jnp.bfloat16),
            jax.ShapeDtypeStruct((_T, _E), jnp.float32),
        ],
    )(x, p["lin"]["W"], p["lin"]["b"].reshape(1, -1),
      p["sw"]["W"], p["sw"]["b"].reshape(1, -1), moe["gate"])

    stb = 512
    shared = pl.pallas_call(
        _shared_body,
        grid=(_T // stb,),
        in_specs=[
            pl.BlockSpec((stb, dout), lambda i: (i, 0)),
            full((hs, dout)), full((hs, dout)), full((dout, hs)),
        ],
        out_specs=pl.BlockSpec((stb, dout), lambda i: (i, 0)),
        out_shape=jax.ShapeDtypeStruct((_T, dout), jnp.float32),
    )(z, moe["sw1"], moe["sw3"], moe["sw2"])

    tbr = 1024
    out = pl.pallas_call(
        _big_routed_body,
        grid=(_T // tbr, _E),
        in_specs=[
            pl.BlockSpec((tbr, dout), lambda i, e: (i, 0)),
            pl.BlockSpec((tbr, _E), lambda i, e: (i, 0)),
            pl.BlockSpec((tbr, dout), lambda i, e: (i, 0)),
            pl.BlockSpec((1, hdim, dout), lambda i, e: (e, 0, 0)),
            pl.BlockSpec((1, hdim, dout), lambda i, e: (e, 0, 0)),
            pl.BlockSpec((1, dout, hdim), lambda i, e: (e, 0, 0)),
        ],
        out_specs=pl.BlockSpec((tbr, dout), lambda i, e: (i, 0)),
        out_shape=jax.ShapeDtypeStruct((_T, dout), jnp.float32),
    )(z, comb, shared,
      moe["w1"].astype(jnp.bfloat16), moe["w3"].astype(jnp.bfloat16),
      moe["w2"].astype(jnp.bfloat16))
    return out


def kernel(x, params):
    skip = _moeff_small(params["enc0"], x, 768, 256)
    b = _moeff_small(params["bott"], skip, 256, 256)
    d = jnp.concatenate([b, skip], axis=1)
    return _moeff_big(params["dec0"], d)


# speedup vs baseline: 1.3132x; 1.0086x over previous
import functools

import jax
import jax.numpy as jnp
from jax.experimental import pallas as pl
from jax.experimental.pallas import tpu as pltpu

_T = 2048
_E = 8
_NG = 4
_GS = _E // _NG


def _hdot(a, b, prec=None):
    return jax.lax.dot_general(
        a, b, (((1,), (1,)), ((), ())), precision=prec,
        preferred_element_type=jnp.float32)


def _bdot(a, b):
    return jax.lax.dot_general(
        a.astype(jnp.bfloat16), b.astype(jnp.bfloat16),
        (((1,), (1,)), ((), ())), preferred_element_type=jnp.float32)


def _silu(v):
    return v * jax.nn.sigmoid(v)


def _routing_combine(z, gate):
    logits = _hdot(z, gate)
    m = jnp.max(logits, axis=1, keepdims=True)
    p = jnp.exp(logits - m)
    scores = p / jnp.sum(p, axis=1, keepdims=True)
    lane = jax.lax.broadcasted_iota(jnp.int32, scores.shape, 1)
    cols = [
        jnp.sum(jnp.where(lane == j, scores, 0.0), axis=1, keepdims=True)
        for j in range(_E)
    ]
    gsc = [jnp.maximum(cols[2 * g], cols[2 * g + 1]) for g in range(_NG)]
    sel = []
    for g in range(_NG):
        rank = jnp.zeros_like(gsc[g])
        for j in range(_NG):
            if j == g:
                continue
            beats = (gsc[j] > gsc[g]) | ((gsc[j] == gsc[g]) & (j < g))
            rank = rank + beats.astype(jnp.float32)
        sel.append((rank < 2.0).astype(jnp.float32))
    group_lane = lane // _GS
    emask = jnp.zeros_like(scores)
    for g in range(_NG):
        emask = emask + jnp.where(group_lane == g, sel[g], 0.0)
    return scores * emask


def _small_layer(dout, x, wl_ref, bl_ref, ws_ref, bs_ref, gate_ref,
                 w1_ref, w3_ref, w2_ref, s1_ref, s3_ref, s2_ref):
    y = _hdot(x, wl_ref[...]) + bl_ref[...]
    h = _hdot(y, ws_ref[...]) + bs_ref[...]
    u = h[:, :dout]
    v = h[:, dout:]
    z = u * _silu(v)
    combine = _routing_combine(z, gate_ref[...])
    a1 = _hdot(z, s1_ref[...])
    a3 = _hdot(z, s3_ref[...])
    acc = _hdot(_silu(a1) * a3, s2_ref[...])
    for e in range(_E):
        h1 = _hdot(z, w1_ref[e])
        h3 = _hdot(z, w3_ref[e])
        eo = _hdot(_silu(h1) * h3, w2_ref[e])
        w = jnp.sum(
            jnp.where(jax.lax.broadcasted_iota(jnp.int32, combine.shape, 1) == e,
                      combine, 0.0), axis=1, keepdims=True)
        acc = acc + w * eo
    return acc


def _enc_bott_body(x_ref, *refs):
    enc_refs = refs[:11]
    bott_refs = refs[11:22]
    skip_ref, b_ref = refs[22], refs[23]
    skip = _small_layer(256, x_ref[...], *enc_refs)
    skip_ref[...] = skip
    b_ref[...] = _small_layer(256, skip, *bott_refs)


def _small_args(p, dout):
    moe = p["moe"]
    return (p["lin"]["W"], p["lin"]["b"].reshape(1, -1),
            p["sw"]["W"], p["sw"]["b"].reshape(1, -1), moe["gate"],
            moe["w1"], moe["w3"], moe["w2"],
            moe["sw1"], moe["sw3"], moe["sw2"])


def _small_specs(din, dout):
    hdim = dout // 2
    hs = dout
    full = lambda shp: pl.BlockSpec(shp, lambda i: (0,) * len(shp))
    return [
        full((dout, din)), full((1, dout)),
        full((2 * dout, dout)), full((1, 2 * dout)),
        full((_E, dout)),
        full((_E, hdim, dout)), full((_E, hdim, dout)),
        full((_E, dout, hdim)),
        full((hs, dout)), full((hs, dout)), full((dout, hs)),
    ]


def _enc_bott(params, x, tb=512):
    out_spec = pl.BlockSpec((tb, 256), lambda i: (i, 0))
    out_shape = jax.ShapeDtypeStruct((_T, 256), jnp.float32)
    return pl.pallas_call(
        _enc_bott_body,
        grid=(_T // tb,),
        in_specs=([pl.BlockSpec((tb, 768), lambda i: (i, 0))]
                  + _small_specs(768, 256) + _small_specs(256, 256)),
        out_specs=[out_spec, out_spec],
        out_shape=[out_shape, out_shape],
    )(x, *_small_args(params["enc0"], 256), *_small_args(params["bott"], 256))


def _big_prelude_body(dout, x_ref, wl_ref, bl_ref, ws_ref, bs_ref, gate_ref,
                      z_ref, comb_ref):
    x = x_ref[...]
    y = _hdot(x, wl_ref[...]) + bl_ref[...]
    h = _hdot(y, ws_ref[...]) + bs_ref[...]
    u = h[:, :dout]
    v = h[:, dout:]
    z = u * _silu(v)
    comb_ref[...] = _routing_combine(z, gate_ref[...])
    z_ref[...] = z.astype(jnp.bfloat16)


def _shared_body(z_ref, s1_ref, s3_ref, s2_ref, out_ref):
    z = z_ref[...].astype(jnp.float32)
    a1 = _hdot(z, s1_ref[...])
    a3 = _hdot(z, s3_ref[...])
    out_ref[...] = _hdot(_silu(a1) * a3, s2_ref[...])


def _big_routed_body(z_ref, comb_ref, shared_ref, w1_ref, w3_ref, w2_ref,
                     out_ref):
    e = pl.program_id(1)

    @pl.when(e == 0)
    def _():
        out_ref[...] = shared_ref[...]

    zb = z_ref[...]
    h1 = _bdot(zb, w1_ref[0])
    h3 = _bdot(zb, w3_ref[0])
    eo = _bdot(_silu(h1) * h3, w2_ref[0])
    comb = comb_ref[...]
    w = jnp.sum(
        jnp.where(jax.lax.broadcasted_iota(jnp.int32, comb.shape, 1) == e,
                  comb, 0.0), axis=1, keepdims=True)
    out_ref[...] = out_ref[...] + w * eo


def _moeff_big(p, x):
    din, dout, hdim = 512, 1536, 768
    hs = dout
    moe = p["moe"]
    full = lambda shp: pl.BlockSpec(shp, lambda i: (0,) * len(shp))
    tb = 512
    z, comb = pl.pallas_call(
        functools.partial(_big_prelude_body, dout),
        grid=(_T // tb,),
        in_specs=[
            pl.BlockSpec((tb, din), lambda i: (i, 0)),
            full((dout, din)), full((1, dout)),
            full((2 * dout, dout)), full((1, 2 * dout)),
            full((_E, dout)),
        ],
        out_specs=[
            pl.BlockSpec((tb, dout), lambda i: (i, 0)),
            pl.BlockSpec((tb, _E), lambda i: (i, 0)),
        ],
        out_shape=[
            jax.ShapeDtypeStruct((_T, dout), jnp.bfloat16),
            jax.ShapeDtypeStruct((_T, _E), jnp.float32),
        ],
    )(x, p["lin"]["W"], p["lin"]["b"].reshape(1, -1),
      p["sw"]["W"], p["sw"]["b"].reshape(1, -1), moe["gate"])

    stb = 512
    shared = pl.pallas_call(
        _shared_body,
        grid=(_T // stb,),
        in_specs=[
            pl.BlockSpec((stb, dout), lambda i: (i, 0)),
            full((hs, dout)), full((hs, dout)), full((dout, hs)),
        ],
        out_specs=pl.BlockSpec((stb, dout), lambda i: (i, 0)),
        out_shape=jax.ShapeDtypeStruct((_T, dout), jnp.float32),
    )(z, moe["sw1"], moe["sw3"], moe["sw2"])

    tbr = 1024
    out = pl.pallas_call(
        _big_routed_body,
        grid=(_T // tbr, _E),
        in_specs=[
            pl.BlockSpec((tbr, dout), lambda i, e: (i, 0)),
            pl.BlockSpec((tbr, _E), lambda i, e: (i, 0)),
            pl.BlockSpec((tbr, dout), lambda i, e: (i, 0)),
            pl.BlockSpec((1, hdim, dout), lambda i, e: (e, 0, 0)),
            pl.BlockSpec((1, hdim, dout), lambda i, e: (e, 0, 0)),
            pl.BlockSpec((1, dout, hdim), lambda i, e: (e, 0, 0)),
        ],
        out_specs=pl.BlockSpec((tbr, dout), lambda i, e: (i, 0)),
        out_shape=jax.ShapeDtypeStruct((_T, dout), jnp.float32),
    )(z, comb, shared,
      moe["w1"].astype(jnp.bfloat16), moe["w3"].astype(jnp.bfloat16),
      moe["w2"].astype(jnp.bfloat16))
    return out


def kernel(x, params):
    skip, b = _enc_bott(params, x)
    d = jnp.concatenate([b, skip], axis=1)
    return _moeff_big(params["dec0"], d)
